# Initial kernel scaffold; baseline (speedup 1.0000x reference)
import jax
import jax.numpy as jnp
from jax.experimental import pallas as pl


def kernel(node_out, a, edge_out, past_index, futur_index, params):
    raise NotImplementedError("write your pallas kernel here")


# speedup vs baseline: 1.2841x; 1.2841x over previous
import functools

import jax
import jax.numpy as jnp
from jax import lax
from jax.experimental import pallas as pl
from jax.experimental.pallas import tpu as pltpu
import jax.experimental.pallas.tpu_sc as plsc

NN = 10000
EE = 160000
DD = 128
DEDGE = 16

NC = 2
NS = 16
NW = NC * NS
CHUNK = 128
NCHUNKS = EE // CHUNK
FULL_ROUNDS = NCHUNKS // NW
REM = NCHUNKS - FULL_ROUNDS * NW
ROWS_PER_SUBCORE = 640
NPAD = ROWS_PER_SUBCORE * NS

f32 = jnp.float32


def _gather_body(x_hbm, src_hbm, dst_hbm, xs_hbm, xd_hbm,
                 idx_s, idx_d, rows_s, rows_d, sem_s, sem_d):
    c = lax.axis_index("c")
    s = lax.axis_index("s")
    wid = s * NC + c

    def do_chunk(j):
        off = pl.multiple_of(j * CHUNK, CHUNK)
        pltpu.sync_copy(src_hbm.at[pl.ds(off, CHUNK)], idx_s)
        pltpu.sync_copy(dst_hbm.at[pl.ds(off, CHUNK)], idx_d)
        cp1 = pltpu.async_copy(x_hbm.at[idx_s], rows_s, sem_s)
        cp2 = pltpu.async_copy(x_hbm.at[idx_d], rows_d, sem_d)
        cp1.wait()
        cp2.wait()
        pltpu.sync_copy(rows_s, xs_hbm.at[pl.ds(off, CHUNK)])
        pltpu.sync_copy(rows_d, xd_hbm.at[pl.ds(off, CHUNK)])

    def body(k, carry):
        do_chunk(k * NW + wid)
        return carry

    lax.fori_loop(0, FULL_ROUNDS, body, 0)

    @pl.when(wid < REM)
    def _():
        do_chunk(FULL_ROUNDS * NW + wid)


def _gather_call(x, src, dst):
    fn = pl.kernel(
        _gather_body,
        out_type=(jax.ShapeDtypeStruct((EE, DD), f32),
                  jax.ShapeDtypeStruct((EE, DD), f32)),
        mesh=plsc.VectorSubcoreMesh(core_axis_name="c", subcore_axis_name="s"),
        scratch_types=[
            pltpu.VMEM((CHUNK,), jnp.int32),
            pltpu.VMEM((CHUNK,), jnp.int32),
            pltpu.VMEM((CHUNK, DD), f32),
            pltpu.VMEM((CHUNK, DD), f32),
            pltpu.SemaphoreType.DMA,
            pltpu.SemaphoreType.DMA,
        ],
    )
    return fn(x, src, dst)


def _scatter_body(m_hbm, idx_hbm, z_hbm, out_hbm,
                  accum, idxv, rows, sem):
    c = lax.axis_index("c")
    s = lax.axis_index("s")
    wid = s * NC + c
    row0 = s * ROWS_PER_SUBCORE

    pltpu.sync_copy(z_hbm, accum.at[pl.ds(row0, ROWS_PER_SUBCORE)])
    plsc.subcore_barrier()

    def do_chunk(j):
        off = pl.multiple_of(j * CHUNK, CHUNK)
        pltpu.sync_copy(idx_hbm.at[pl.ds(off, CHUNK)], idxv)
        pltpu.sync_copy(m_hbm.at[pl.ds(off, CHUNK)], rows)
        pltpu.sync_copy(rows, accum.at[idxv], add=True)

    def body(k, carry):
        do_chunk(k * NW + wid)
        return carry

    lax.fori_loop(0, FULL_ROUNDS, body, 0)

    @pl.when(wid < REM)
    def _():
        do_chunk(FULL_ROUNDS * NW + wid)

    plsc.subcore_barrier()
    pltpu.sync_copy(accum.at[pl.ds(row0, ROWS_PER_SUBCORE)],
                    out_hbm.at[c, pl.ds(row0, ROWS_PER_SUBCORE)])


def _scatter_call(m, idx, zeros):
    fn = pl.kernel(
        _scatter_body,
        out_type=jax.ShapeDtypeStruct((NC, NPAD, DD), f32),
        mesh=plsc.VectorSubcoreMesh(core_axis_name="c", subcore_axis_name="s"),
        scratch_types=[
            pltpu.VMEM_SHARED((NPAD, DD), f32),
            pltpu.VMEM((CHUNK,), jnp.int32),
            pltpu.VMEM((CHUNK, DD), f32),
            pltpu.SemaphoreType.DMA,
        ],
    )
    return fn(m, idx, zeros)


BE = 640


def _edge_flow_body(xs, xd, e,
                    w0s, w0d, w0e, b0, w1, b1, w2, b2,
                    ax, ae, a0, a1w, a1b, a2w, a2b,
                    bx, be_, bb0, b1w, b1b, b2w, b2b,
                    e_new, m_in, m_out):
    dot = functools.partial(jnp.dot, preferred_element_type=f32)
    xs_v = xs[...]
    xd_v = xd[...]
    h = jax.nn.relu(dot(xs_v, w0s[...]) + dot(xd_v, w0d[...])
                    + dot(e[...], w0e[...]) + b0[...])
    h = jax.nn.relu(dot(h, w1[...]) + b1[...])
    en = dot(h, w2[...]) + b2[...]
    e_new[...] = en

    fi = jax.nn.relu(dot(xs_v, ax[...]) + dot(en, ae[...]) + a0[...])
    fi = jax.nn.relu(dot(fi, a1w[...]) + a1b[...])
    m_in[...] = dot(fi, a2w[...]) + a2b[...]

    fo = jax.nn.relu(dot(xd_v, bx[...]) + dot(en, be_[...]) + bb0[...])
    fo = jax.nn.relu(dot(fo, b1w[...]) + b1b[...])
    m_out[...] = dot(fo, b2w[...]) + b2b[...]


def _edge_flow_call(xs, xd, e, weights):
    data_specs = [
        pl.BlockSpec((BE, DD), lambda i: (i, 0)),
        pl.BlockSpec((BE, DD), lambda i: (i, 0)),
        pl.BlockSpec((BE, DEDGE), lambda i: (i, 0)),
    ]
    w_specs = [pl.BlockSpec(w.shape, lambda i: (0,) * w.ndim) for w in weights]
    out_shapes = (jax.ShapeDtypeStruct((EE, DEDGE), f32),
                  jax.ShapeDtypeStruct((EE, DD), f32),
                  jax.ShapeDtypeStruct((EE, DD), f32))
    out_specs = (pl.BlockSpec((BE, DEDGE), lambda i: (i, 0)),
                 pl.BlockSpec((BE, DD), lambda i: (i, 0)),
                 pl.BlockSpec((BE, DD), lambda i: (i, 0)))
    return pl.pallas_call(
        _edge_flow_body,
        grid=(EE // BE,),
        in_specs=data_specs + w_specs,
        out_specs=out_specs,
        out_shape=out_shapes,
        compiler_params=pltpu.CompilerParams(
            dimension_semantics=("arbitrary",)),
    )(xs, xd, e, *weights)


BN = 1000


def _node_body(pp, pf, w0a, w0b, b0, w1, b1, w2, b2, x_new):
    dot = functools.partial(jnp.dot, preferred_element_type=f32)
    agg_p = pp[0] + pp[1]
    agg_f = pf[0] + pf[1]
    h = jax.nn.relu(dot(agg_p, w0a[...]) + dot(agg_f, w0b[...]) + b0[...])
    h = jax.nn.relu(dot(h, w1[...]) + b1[...])
    x_new[...] = dot(h, w2[...]) + b2[...]


def _node_call(pp, pf, weights):
    data_specs = [
        pl.BlockSpec((NC, BN, DD), lambda i: (0, i, 0)),
        pl.BlockSpec((NC, BN, DD), lambda i: (0, i, 0)),
    ]
    w_specs = [pl.BlockSpec(w.shape, lambda i: (0,) * w.ndim) for w in weights]
    return pl.pallas_call(
        _node_body,
        grid=(NN // BN,),
        in_specs=data_specs + w_specs,
        out_specs=pl.BlockSpec((BN, DD), lambda i: (i, 0)),
        out_shape=jax.ShapeDtypeStruct((NN, DD), f32),
        compiler_params=pltpu.CompilerParams(
            dimension_semantics=("arbitrary",)),
    )(pp, pf, *weights)


BC = 2000


def _cls_body(e, w0, b0, w1, b1, w2, b2, out):
    dot = functools.partial(jnp.dot, preferred_element_type=f32)
    prelu = lambda v: jnp.where(v >= 0, v, 0.25 * v)
    h = prelu(dot(e[...], w0[...]) + b0[...])
    h = prelu(dot(h, w1[...]) + b1[...])
    out[...] = jax.nn.sigmoid(dot(h, w2[...]) + b2[...])


def _cls_call(e, weights):
    data_specs = [pl.BlockSpec((BC, DEDGE), lambda i: (i, 0))]
    w_specs = [pl.BlockSpec(w.shape, lambda i: (0,) * w.ndim) for w in weights]
    return pl.pallas_call(
        _cls_body,
        grid=(EE // BC,),
        in_specs=data_specs + w_specs,
        out_specs=pl.BlockSpec((BC, 1), lambda i: (i, 0)),
        out_shape=jax.ShapeDtypeStruct((EE, 1), f32),
        compiler_params=pltpu.CompilerParams(
            dimension_semantics=("arbitrary",)),
    )(e, *weights)


def kernel(node_out, a, edge_out, past_index, futur_index, params):
    src = a[0]
    dst = a[1]

    pe = params["edge_mlp"]
    w0 = pe["W"][0]
    ef_weights = [
        w0[:DD], w0[DD:2 * DD], w0[2 * DD:], pe["b"][0][None, :],
        pe["W"][1], pe["b"][1][None, :], pe["W"][2], pe["b"][2][None, :],
    ]
    for nm in ("flow_in", "flow_out"):
        p = params[nm]
        v0 = p["W"][0]
        ef_weights += [
            v0[:DD], v0[DD:], p["b"][0][None, :],
            p["W"][1], p["b"][1][None, :], p["W"][2], p["b"][2][None, :],
        ]

    pn = params["node_mlp"]
    n0 = pn["W"][0]
    n_weights = [n0[:DD], n0[DD:], pn["b"][0][None, :],
                 pn["W"][1], pn["b"][1][None, :],
                 pn["W"][2], pn["b"][2][None, :]]

    pc = params["edge_classifier"]
    c_weights = [pc["W"][0], pc["b"][0][None, :],
                 pc["W"][1], pc["b"][1][None, :],
                 pc["W"][2], pc["b"][2][None, :]]

    zeros = jnp.zeros((ROWS_PER_SUBCORE, DD), f32)
    x = node_out
    e = edge_out
    for _ in range(4):
        xs, xd = _gather_call(x, src, dst)
        e, m_in, m_out = _edge_flow_call(xs, xd, e, ef_weights)
        pp = _scatter_call(m_in, past_index, zeros)
        pf = _scatter_call(m_out, futur_index, zeros)
        x = _node_call(pp, pf, n_weights)
    return _cls_call(e, c_weights)


# speedup vs baseline: 1.4364x; 1.1186x over previous
import functools

import jax
import jax.numpy as jnp
from jax import lax
from jax.experimental import pallas as pl
from jax.experimental.pallas import tpu as pltpu
import jax.experimental.pallas.tpu_sc as plsc

NN = 10000
EE = 160000
DD = 128
DEDGE = 16

NC = 2
NS = 16
NW = NC * NS
CHUNK = 128
NCHUNKS = EE // CHUNK
FULL_ROUNDS = NCHUNKS // NW
REM = NCHUNKS - FULL_ROUNDS * NW
ROWS_PER_SUBCORE = 640
NPAD = ROWS_PER_SUBCORE * NS

f32 = jnp.float32


def _gather_body(x_hbm, src_hbm, dst_hbm, xs_hbm, xd_hbm,
                 idx_s, idx_d, rows_s, rows_d, sem_s, sem_d):
    c = lax.axis_index("c")
    s = lax.axis_index("s")
    wid = s * NC + c

    def do_chunk(j):
        off = pl.multiple_of(j * CHUNK, CHUNK)
        pltpu.sync_copy(src_hbm.at[pl.ds(off, CHUNK)], idx_s)
        pltpu.sync_copy(dst_hbm.at[pl.ds(off, CHUNK)], idx_d)
        cp1 = pltpu.async_copy(x_hbm.at[idx_s], rows_s, sem_s)
        cp2 = pltpu.async_copy(x_hbm.at[idx_d], rows_d, sem_d)
        cp1.wait()
        cp2.wait()
        pltpu.sync_copy(rows_s, xs_hbm.at[pl.ds(off, CHUNK)])
        pltpu.sync_copy(rows_d, xd_hbm.at[pl.ds(off, CHUNK)])

    def body(k, carry):
        do_chunk(k * NW + wid)
        return carry

    lax.fori_loop(0, FULL_ROUNDS, body, 0)

    @pl.when(wid < REM)
    def _():
        do_chunk(FULL_ROUNDS * NW + wid)


def _gather_call(x, src, dst):
    fn = pl.kernel(
        _gather_body,
        out_type=(jax.ShapeDtypeStruct((EE, DD), f32),
                  jax.ShapeDtypeStruct((EE, DD), f32)),
        mesh=plsc.VectorSubcoreMesh(core_axis_name="c", subcore_axis_name="s"),
        scratch_types=[
            pltpu.VMEM((CHUNK,), jnp.int32),
            pltpu.VMEM((CHUNK,), jnp.int32),
            pltpu.VMEM((CHUNK, DD), f32),
            pltpu.VMEM((CHUNK, DD), f32),
            pltpu.SemaphoreType.DMA,
            pltpu.SemaphoreType.DMA,
        ],
    )
    return fn(x, src, dst)


SC_ROUNDS = NCHUNKS // NS
SC_REM = NCHUNKS - SC_ROUNDS * NS


def _scatter_body(min_hbm, mout_hbm, pidx_hbm, fidx_hbm, z_hbm, out_hbm,
                  accum, idxv, rows, sem):
    c = lax.axis_index("c")
    s = lax.axis_index("s")
    row0 = s * ROWS_PER_SUBCORE

    pltpu.sync_copy(z_hbm, accum.at[pl.ds(row0, ROWS_PER_SUBCORE)])
    plsc.subcore_barrier()

    def sweep(m_hbm, idx_hbm):
        def do_chunk(j):
            off = pl.multiple_of(j * CHUNK, CHUNK)
            pltpu.sync_copy(idx_hbm.at[pl.ds(off, CHUNK)], idxv)
            pltpu.sync_copy(m_hbm.at[pl.ds(off, CHUNK)], rows)
            pltpu.sync_copy(rows, accum.at[idxv], add=True)

        def body(k, carry):
            do_chunk(k * NS + s)
            return carry

        lax.fori_loop(0, SC_ROUNDS, body, 0)

        @pl.when(s < SC_REM)
        def _():
            do_chunk(SC_ROUNDS * NS + s)

    @pl.when(c == 0)
    def _():
        sweep(min_hbm, pidx_hbm)

    @pl.when(c == 1)
    def _():
        sweep(mout_hbm, fidx_hbm)

    plsc.subcore_barrier()
    pltpu.sync_copy(accum.at[pl.ds(row0, ROWS_PER_SUBCORE)],
                    out_hbm.at[c, pl.ds(row0, ROWS_PER_SUBCORE)])


def _scatter_call(m_in, m_out, pidx, fidx, zeros):
    fn = pl.kernel(
        _scatter_body,
        out_type=jax.ShapeDtypeStruct((NC, NPAD, DD), f32),
        mesh=plsc.VectorSubcoreMesh(core_axis_name="c", subcore_axis_name="s"),
        scratch_types=[
            pltpu.VMEM_SHARED((NPAD, DD), f32),
            pltpu.VMEM((CHUNK,), jnp.int32),
            pltpu.VMEM((CHUNK, DD), f32),
            pltpu.SemaphoreType.DMA,
        ],
    )
    return fn(m_in, m_out, pidx, fidx, zeros)


BE = 640


def _edge_flow_body(xs, xd, e,
                    w0s, w0d, w0e, b0, w1, b1, w2, b2,
                    ax, ae, a0, a1w, a1b, a2w, a2b,
                    bx, be_, bb0, b1w, b1b, b2w, b2b,
                    e_new, m_in, m_out):
    dot = functools.partial(jnp.dot, preferred_element_type=f32)
    xs_v = xs[...]
    xd_v = xd[...]
    h = jax.nn.relu(dot(xs_v, w0s[...]) + dot(xd_v, w0d[...])
                    + dot(e[...], w0e[...]) + b0[...])
    h = jax.nn.relu(dot(h, w1[...]) + b1[...])
    en = dot(h, w2[...]) + b2[...]
    e_new[...] = en

    fi = jax.nn.relu(dot(xs_v, ax[...]) + dot(en, ae[...]) + a0[...])
    fi = jax.nn.relu(dot(fi, a1w[...]) + a1b[...])
    m_in[...] = dot(fi, a2w[...]) + a2b[...]

    fo = jax.nn.relu(dot(xd_v, bx[...]) + dot(en, be_[...]) + bb0[...])
    fo = jax.nn.relu(dot(fo, b1w[...]) + b1b[...])
    m_out[...] = dot(fo, b2w[...]) + b2b[...]


def _edge_flow_call(xs, xd, e, weights):
    data_specs = [
        pl.BlockSpec((BE, DD), lambda i: (i, 0)),
        pl.BlockSpec((BE, DD), lambda i: (i, 0)),
        pl.BlockSpec((BE, DEDGE), lambda i: (i, 0)),
    ]
    w_specs = [pl.BlockSpec(w.shape, lambda i: (0,) * w.ndim) for w in weights]
    out_shapes = (jax.ShapeDtypeStruct((EE, DEDGE), f32),
                  jax.ShapeDtypeStruct((EE, DD), f32),
                  jax.ShapeDtypeStruct((EE, DD), f32))
    out_specs = (pl.BlockSpec((BE, DEDGE), lambda i: (i, 0)),
                 pl.BlockSpec((BE, DD), lambda i: (i, 0)),
                 pl.BlockSpec((BE, DD), lambda i: (i, 0)))
    return pl.pallas_call(
        _edge_flow_body,
        grid=(EE // BE,),
        in_specs=data_specs + w_specs,
        out_specs=out_specs,
        out_shape=out_shapes,
        compiler_params=pltpu.CompilerParams(
            dimension_semantics=("arbitrary",)),
    )(xs, xd, e, *weights)


BN = 1000


def _node_body(agg, w0a, w0b, b0, w1, b1, w2, b2, x_new):
    dot = functools.partial(jnp.dot, preferred_element_type=f32)
    h = jax.nn.relu(dot(agg[0], w0a[...]) + dot(agg[1], w0b[...]) + b0[...])
    h = jax.nn.relu(dot(h, w1[...]) + b1[...])
    x_new[...] = dot(h, w2[...]) + b2[...]


def _node_call(agg, weights):
    data_specs = [
        pl.BlockSpec((NC, BN, DD), lambda i: (0, i, 0)),
    ]
    w_specs = [pl.BlockSpec(w.shape, lambda i: (0,) * w.ndim) for w in weights]
    return pl.pallas_call(
        _node_body,
        grid=(NN // BN,),
        in_specs=data_specs + w_specs,
        out_specs=pl.BlockSpec((BN, DD), lambda i: (i, 0)),
        out_shape=jax.ShapeDtypeStruct((NN, DD), f32),
        compiler_params=pltpu.CompilerParams(
            dimension_semantics=("arbitrary",)),
    )(agg, *weights)


def _edge_cls_body(xs, xd, e,
                   w0s, w0d, w0e, b0, w1, b1, w2, b2,
                   c0, cb0, c1, cb1, c2, cb2, out):
    dot = functools.partial(jnp.dot, preferred_element_type=f32)
    prelu = lambda v: jnp.where(v >= 0, v, 0.25 * v)
    h = jax.nn.relu(dot(xs[...], w0s[...]) + dot(xd[...], w0d[...])
                    + dot(e[...], w0e[...]) + b0[...])
    h = jax.nn.relu(dot(h, w1[...]) + b1[...])
    en = dot(h, w2[...]) + b2[...]
    g = prelu(dot(en, c0[...]) + cb0[...])
    g = prelu(dot(g, c1[...]) + cb1[...])
    out[...] = jax.nn.sigmoid(dot(g, c2[...]) + cb2[...])


def _edge_cls_call(xs, xd, e, weights):
    data_specs = [
        pl.BlockSpec((BE, DD), lambda i: (i, 0)),
        pl.BlockSpec((BE, DD), lambda i: (i, 0)),
        pl.BlockSpec((BE, DEDGE), lambda i: (i, 0)),
    ]
    w_specs = [pl.BlockSpec(w.shape, lambda i: (0,) * w.ndim) for w in weights]
    return pl.pallas_call(
        _edge_cls_body,
        grid=(EE // BE,),
        in_specs=data_specs + w_specs,
        out_specs=pl.BlockSpec((BE, 1), lambda i: (i, 0)),
        out_shape=jax.ShapeDtypeStruct((EE, 1), f32),
        compiler_params=pltpu.CompilerParams(
            dimension_semantics=("arbitrary",)),
    )(xs, xd, e, *weights)


def kernel(node_out, a, edge_out, past_index, futur_index, params):
    src = a[0]
    dst = a[1]

    pe = params["edge_mlp"]
    w0 = pe["W"][0]
    ef_weights = [
        w0[:DD], w0[DD:2 * DD], w0[2 * DD:], pe["b"][0][None, :],
        pe["W"][1], pe["b"][1][None, :], pe["W"][2], pe["b"][2][None, :],
    ]
    for nm in ("flow_in", "flow_out"):
        p = params[nm]
        v0 = p["W"][0]
        ef_weights += [
            v0[:DD], v0[DD:], p["b"][0][None, :],
            p["W"][1], p["b"][1][None, :], p["W"][2], p["b"][2][None, :],
        ]

    pn = params["node_mlp"]
    n0 = pn["W"][0]
    n_weights = [n0[:DD], n0[DD:], pn["b"][0][None, :],
                 pn["W"][1], pn["b"][1][None, :],
                 pn["W"][2], pn["b"][2][None, :]]

    pc = params["edge_classifier"]
    c_weights = [pc["W"][0], pc["b"][0][None, :],
                 pc["W"][1], pc["b"][1][None, :],
                 pc["W"][2], pc["b"][2][None, :]]

    zeros = jnp.zeros((ROWS_PER_SUBCORE, DD), f32)
    x = node_out
    e = edge_out
    for _ in range(3):
        xs, xd = _gather_call(x, src, dst)
        e, m_in, m_out = _edge_flow_call(xs, xd, e, ef_weights)
        agg = _scatter_call(m_in, m_out, past_index, futur_index, zeros)
        x = _node_call(agg, n_weights)
    xs, xd = _gather_call(x, src, dst)
    return _edge_cls_call(xs, xd, e, ef_weights[:8] + c_weights)


# speedup vs baseline: 1.4440x; 1.0052x over previous
import functools

import jax
import jax.numpy as jnp
from jax import lax
from jax.experimental import pallas as pl
from jax.experimental.pallas import tpu as pltpu
import jax.experimental.pallas.tpu_sc as plsc

NN = 10000
EE = 160000
DD = 128
DEDGE = 16

NC = 2
NS = 16
NW = NC * NS
CHUNK = 128
NCHUNKS = EE // CHUNK
FULL_ROUNDS = NCHUNKS // NW
REM = NCHUNKS - FULL_ROUNDS * NW
ROWS_PER_SUBCORE = 640
NPAD = ROWS_PER_SUBCORE * NS

f32 = jnp.float32
MM_PRECISION = None


def _gather_body(x_hbm, src_hbm, dst_hbm, xs_hbm, xd_hbm,
                 idx_s, idx_d, rows_s, rows_d, sem_s, sem_d):
    c = lax.axis_index("c")
    s = lax.axis_index("s")
    wid = s * NC + c

    def do_chunk(j):
        off = pl.multiple_of(j * CHUNK, CHUNK)
        pltpu.sync_copy(src_hbm.at[pl.ds(off, CHUNK)], idx_s)
        pltpu.sync_copy(dst_hbm.at[pl.ds(off, CHUNK)], idx_d)
        cp1 = pltpu.async_copy(x_hbm.at[idx_s], rows_s, sem_s)
        cp2 = pltpu.async_copy(x_hbm.at[idx_d], rows_d, sem_d)
        cp1.wait()
        cp2.wait()
        pltpu.sync_copy(rows_s, xs_hbm.at[pl.ds(off, CHUNK)])
        pltpu.sync_copy(rows_d, xd_hbm.at[pl.ds(off, CHUNK)])

    def body(k, carry):
        do_chunk(k * NW + wid)
        return carry

    lax.fori_loop(0, FULL_ROUNDS, body, 0)

    @pl.when(wid < REM)
    def _():
        do_chunk(FULL_ROUNDS * NW + wid)


def _gather_call(x, src, dst):
    fn = pl.kernel(
        _gather_body,
        out_type=(jax.ShapeDtypeStruct((EE, DD), f32),
                  jax.ShapeDtypeStruct((EE, DD), f32)),
        mesh=plsc.VectorSubcoreMesh(core_axis_name="c", subcore_axis_name="s"),
        scratch_types=[
            pltpu.VMEM((CHUNK,), jnp.int32),
            pltpu.VMEM((CHUNK,), jnp.int32),
            pltpu.VMEM((CHUNK, DD), f32),
            pltpu.VMEM((CHUNK, DD), f32),
            pltpu.SemaphoreType.DMA,
            pltpu.SemaphoreType.DMA,
        ],
    )
    return fn(x, src, dst)


DUMP = NPAD


def _scatter_body(min_hbm, mout_hbm, sp_hbm, pp_hbm, sf_hbm, pf_hbm,
                  bnd_hbm, z_hbm, out_hbm,
                  accum, idxv, permv, rows, bndv, sem):
    c = lax.axis_index("c")
    s = lax.axis_index("s")
    row0 = s * ROWS_PER_SUBCORE

    pltpu.sync_copy(z_hbm, accum.at[pl.ds(row0, ROWS_PER_SUBCORE)])

    @pl.when(s == 0)
    def _():
        pltpu.sync_copy(z_hbm.at[pl.ds(0, 8)], accum.at[pl.ds(DUMP, 8)])

    pltpu.sync_copy(bnd_hbm.at[c, s], bndv)
    plsc.subcore_barrier()

    def sweep(m_hbm, sidx_hbm, perm_hbm):
        bv = bndv[...]
        astart = bv[0]
        end = bv[1]

        def do_chunk(i, carry):
            base = pl.multiple_of(astart + i * CHUNK, 8)
            pltpu.sync_copy(sidx_hbm.at[pl.ds(base, CHUNK)], idxv)
            pltpu.sync_copy(perm_hbm.at[pl.ds(base, CHUNK)], permv)
            for j in range(CHUNK // 16):
                v = idxv[pl.ds(j * 16, 16)]
                pos = base + j * 16 + lax.broadcasted_iota(jnp.int32, (16,), 0)
                valid = ((pos < end) & (v >= row0)
                         & (v < row0 + ROWS_PER_SUBCORE))
                idxv[pl.ds(j * 16, 16)] = jnp.where(valid, v, DUMP)
            pltpu.async_copy(m_hbm.at[permv], rows, sem).wait()
            pltpu.sync_copy(rows, accum.at[idxv], add=True)
            return carry

        nch = lax.div(end - astart + (CHUNK - 1), CHUNK)
        lax.fori_loop(0, nch, do_chunk, 0)

    @pl.when(c == 0)
    def _():
        sweep(min_hbm, sp_hbm, pp_hbm)

    @pl.when(c == 1)
    def _():
        sweep(mout_hbm, sf_hbm, pf_hbm)

    plsc.subcore_barrier()
    pltpu.sync_copy(accum.at[pl.ds(row0, ROWS_PER_SUBCORE)],
                    out_hbm.at[c, pl.ds(row0, ROWS_PER_SUBCORE)])


def _scatter_call(m_in, m_out, sp, pp, sf, pf, bnd, zeros):
    fn = pl.kernel(
        _scatter_body,
        out_type=jax.ShapeDtypeStruct((NC, NPAD, DD), f32),
        mesh=plsc.VectorSubcoreMesh(core_axis_name="c", subcore_axis_name="s"),
        scratch_types=[
            pltpu.VMEM_SHARED((NPAD + 8, DD), f32),
            pltpu.VMEM((CHUNK,), jnp.int32),
            pltpu.VMEM((CHUNK,), jnp.int32),
            pltpu.VMEM((CHUNK, DD), f32),
            pltpu.VMEM((16,), jnp.int32),
            pltpu.SemaphoreType.DMA,
        ],
    )
    return fn(m_in, m_out, sp, pp, sf, pf, bnd, zeros)


def _sort_plan(idx):
    perm = jnp.argsort(idx, stable=True).astype(jnp.int32)
    sidx = idx[perm].astype(jnp.int32)
    edges = jnp.arange(0, NPAD + 1, ROWS_PER_SUBCORE)
    cuts = jnp.searchsorted(sidx, edges).astype(jnp.int32)
    starts = (cuts[:-1] // 8) * 8
    ends = cuts[1:]
    bnd = jnp.zeros((NS, 16), jnp.int32)
    bnd = bnd.at[:, 0].set(starts).at[:, 1].set(ends)
    pad = jnp.zeros((CHUNK,), jnp.int32)
    return (jnp.concatenate([sidx, pad]), jnp.concatenate([perm, pad]), bnd)


BE = 640


def _edge_flow_body(xs, xd, e,
                    w0, b0, w1, b1, w2, b2,
                    a0w, a0b, a1w, a1b, a2w, a2b,
                    b0w, b0b, b1w, b1b, b2w, b2b,
                    e_new, m_in, m_out):
    dot = functools.partial(jnp.dot, preferred_element_type=f32, precision=MM_PRECISION)
    xs_v = xs[...]
    xd_v = xd[...]
    h = jax.nn.relu(dot(jnp.concatenate([xs_v, xd_v, e[...]], axis=-1),
                        w0[...]) + b0[...])
    h = jax.nn.relu(dot(h, w1[...]) + b1[...])
    en = dot(h, w2[...]) + b2[...]
    e_new[...] = en

    fi = jax.nn.relu(dot(jnp.concatenate([xs_v, en], axis=-1), a0w[...])
                     + a0b[...])
    fi = jax.nn.relu(dot(fi, a1w[...]) + a1b[...])
    m_in[...] = dot(fi, a2w[...]) + a2b[...]

    fo = jax.nn.relu(dot(jnp.concatenate([xd_v, en], axis=-1), b0w[...])
                     + b0b[...])
    fo = jax.nn.relu(dot(fo, b1w[...]) + b1b[...])
    m_out[...] = dot(fo, b2w[...]) + b2b[...]


def _edge_flow_call(xs, xd, e, weights):
    data_specs = [
        pl.BlockSpec((BE, DD), lambda i: (i, 0)),
        pl.BlockSpec((BE, DD), lambda i: (i, 0)),
        pl.BlockSpec((BE, DEDGE), lambda i: (i, 0)),
    ]
    w_specs = [pl.BlockSpec(w.shape, lambda i: (0,) * w.ndim) for w in weights]
    out_shapes = (jax.ShapeDtypeStruct((EE, DEDGE), f32),
                  jax.ShapeDtypeStruct((EE, DD), f32),
                  jax.ShapeDtypeStruct((EE, DD), f32))
    out_specs = (pl.BlockSpec((BE, DEDGE), lambda i: (i, 0)),
                 pl.BlockSpec((BE, DD), lambda i: (i, 0)),
                 pl.BlockSpec((BE, DD), lambda i: (i, 0)))
    return pl.pallas_call(
        _edge_flow_body,
        grid=(EE // BE,),
        in_specs=data_specs + w_specs,
        out_specs=out_specs,
        out_shape=out_shapes,
        compiler_params=pltpu.CompilerParams(
            dimension_semantics=("arbitrary",)),
    )(xs, xd, e, *weights)


BN = 1000


def _node_body(agg, w0, b0, w1, b1, w2, b2, x_new):
    dot = functools.partial(jnp.dot, preferred_element_type=f32, precision=MM_PRECISION)
    h = jax.nn.relu(dot(jnp.concatenate([agg[0], agg[1]], axis=-1), w0[...])
                    + b0[...])
    h = jax.nn.relu(dot(h, w1[...]) + b1[...])
    x_new[...] = dot(h, w2[...]) + b2[...]


def _node_call(agg, weights):
    data_specs = [
        pl.BlockSpec((NC, BN, DD), lambda i: (0, i, 0)),
    ]
    w_specs = [pl.BlockSpec(w.shape, lambda i: (0,) * w.ndim) for w in weights]
    return pl.pallas_call(
        _node_body,
        grid=(NN // BN,),
        in_specs=data_specs + w_specs,
        out_specs=pl.BlockSpec((BN, DD), lambda i: (i, 0)),
        out_shape=jax.ShapeDtypeStruct((NN, DD), f32),
        compiler_params=pltpu.CompilerParams(
            dimension_semantics=("arbitrary",)),
    )(agg, *weights)


def _edge_cls_body(xs, xd, e,
                   w0, b0, w1, b1, w2, b2,
                   c0, cb0, c1, cb1, c2, cb2, out):
    dot = functools.partial(jnp.dot, preferred_element_type=f32, precision=MM_PRECISION)
    prelu = lambda v: jnp.where(v >= 0, v, 0.25 * v)
    h = jax.nn.relu(dot(jnp.concatenate([xs[...], xd[...], e[...]], axis=-1),
                        w0[...]) + b0[...])
    h = jax.nn.relu(dot(h, w1[...]) + b1[...])
    en = dot(h, w2[...]) + b2[...]
    g = prelu(dot(en, c0[...]) + cb0[...])
    g = prelu(dot(g, c1[...]) + cb1[...])
    out[...] = jax.nn.sigmoid(dot(g, c2[...]) + cb2[...])


def _edge_cls_call(xs, xd, e, weights):
    data_specs = [
        pl.BlockSpec((BE, DD), lambda i: (i, 0)),
        pl.BlockSpec((BE, DD), lambda i: (i, 0)),
        pl.BlockSpec((BE, DEDGE), lambda i: (i, 0)),
    ]
    w_specs = [pl.BlockSpec(w.shape, lambda i: (0,) * w.ndim) for w in weights]
    return pl.pallas_call(
        _edge_cls_body,
        grid=(EE // BE,),
        in_specs=data_specs + w_specs,
        out_specs=pl.BlockSpec((BE, 1), lambda i: (i, 0)),
        out_shape=jax.ShapeDtypeStruct((EE, 1), f32),
        compiler_params=pltpu.CompilerParams(
            dimension_semantics=("arbitrary",)),
    )(xs, xd, e, *weights)


def kernel(node_out, a, edge_out, past_index, futur_index, params):
    src = a[0]
    dst = a[1]

    pe = params["edge_mlp"]
    ef_weights = [
        pe["W"][0], pe["b"][0][None, :],
        pe["W"][1], pe["b"][1][None, :], pe["W"][2], pe["b"][2][None, :],
    ]
    for nm in ("flow_in", "flow_out"):
        p = params[nm]
        ef_weights += [
            p["W"][0], p["b"][0][None, :],
            p["W"][1], p["b"][1][None, :], p["W"][2], p["b"][2][None, :],
        ]

    pn = params["node_mlp"]
    n_weights = [pn["W"][0], pn["b"][0][None, :],
                 pn["W"][1], pn["b"][1][None, :],
                 pn["W"][2], pn["b"][2][None, :]]

    pc = params["edge_classifier"]
    c_weights = [pc["W"][0], pc["b"][0][None, :],
                 pc["W"][1], pc["b"][1][None, :],
                 pc["W"][2], pc["b"][2][None, :]]

    zeros = jnp.zeros((ROWS_PER_SUBCORE, DD), f32)
    sp, pp, bnd_p = _sort_plan(past_index)
    sf, pf, bnd_f = _sort_plan(futur_index)
    bnd = jnp.stack([bnd_p, bnd_f])
    x = node_out
    e = edge_out
    for _ in range(3):
        xs, xd = _gather_call(x, src, dst)
        e, m_in, m_out = _edge_flow_call(xs, xd, e, ef_weights)
        agg = _scatter_call(m_in, m_out, sp, pp, sf, pf, bnd, zeros)
        x = _node_call(agg, n_weights)
    xs, xd = _gather_call(x, src, dst)
    return _edge_cls_call(xs, xd, e, ef_weights[:6] + c_weights)
